# Initial kernel scaffold; baseline (speedup 1.0000x reference)
import jax
import jax.numpy as jnp
from jax.experimental import pallas as pl


def kernel(input, target):
    raise NotImplementedError("write your pallas kernel here")


# speedup vs baseline: 15.7466x; 15.7466x over previous
import functools

import jax
import jax.numpy as jnp
from jax import lax
from jax.experimental import pallas as pl
from jax.experimental.pallas import tpu as pltpu
from jax.experimental.pallas import tpu_sc as plsc

B = 8
N = 512 * 512
BINS = 32
NBINS2 = BINS * BINS

_info = plsc.get_sparse_core_info()
NC = _info.num_cores
NS = _info.num_subcores
NW = NC * NS
L = _info.num_lanes
CHUNK = N // NW


def _sc_hist_body(x_hbm, y_hbm, out_hbm, xv, yv, hist, sem):
    wid = lax.axis_index("c") * NS + lax.axis_index("s")
    base = wid * CHUNK

    zeros = jnp.zeros((L,), jnp.float32)
    ones = jnp.ones((L,), jnp.float32)

    def zero_body(i, _):
        hist[pl.ds(i * L, L)] = zeros
        return 0

    lax.fori_loop(0, (B * NBINS2) // L, zero_body, 0)

    for b in range(B):
        pltpu.sync_copy(x_hbm.at[b, pl.ds(base, CHUNK)], xv)
        pltpu.sync_copy(y_hbm.at[b, pl.ds(base, CHUNK)], yv)
        hbase = b * NBINS2

        def point_body(i, _):
            xr = xv[pl.ds(i * L, L)]
            yr = yv[pl.ds(i * L, L)]
            vx = xr * 16.0 + 16.0
            vy = yr * 16.0 + 16.0
            valid = (vx >= 0.0) & (vx <= 32.0) & (vy >= 0.0) & (vy <= 32.0)
            ix = jnp.minimum(vx.astype(jnp.int32), BINS - 1)
            iy = jnp.minimum(vy.astype(jnp.int32), BINS - 1)
            flat = hbase + ix * BINS + iy
            plsc.addupdate_scatter(hist, [flat], ones, mask=valid)
            return 0

        lax.fori_loop(0, CHUNK // L, point_body, 0)

    pltpu.sync_copy(hist, out_hbm.at[wid])


@functools.partial(
    pl.kernel,
    out_type=jax.ShapeDtypeStruct((NW, B * NBINS2), jnp.float32),
    mesh=plsc.VectorSubcoreMesh(core_axis_name="c", subcore_axis_name="s"),
    compiler_params=pltpu.CompilerParams(needs_layout_passes=False),
    scratch_types=[
        pltpu.VMEM((CHUNK,), jnp.float32),
        pltpu.VMEM((CHUNK,), jnp.float32),
        pltpu.VMEM((B * NBINS2,), jnp.float32),
        pltpu.SemaphoreType.DMA,
    ],
)
def _sc_hist(x_hbm, y_hbm, out_hbm, xv, yv, hist, sem):
    _sc_hist_body(x_hbm, y_hbm, out_hbm, xv, yv, hist, sem)


def _tc_mi_body(parts_ref, out_ref):
    h = jnp.sum(parts_ref[...], axis=0)
    px = jnp.sum(h, axis=2)
    py = jnp.sum(h, axis=1)
    tot = jnp.sum(h, axis=(1, 2), keepdims=True)
    hn = h / tot
    pxn = px / jnp.sum(px, axis=1, keepdims=True)
    pyn = py / jnp.sum(py, axis=1, keepdims=True)
    pxy = pxn[:, :, None] * pyn[:, None, :]
    mask = (hn > 0) & (pxn[:, :, None] > 0) & (pyn[:, None, :] > 0)
    safe_ratio = jnp.where(mask, hn / jnp.where(mask, pxy, 1.0), 1.0)
    mi = jnp.sum(jnp.where(mask, hn * jnp.log(safe_ratio), 0.0))
    out_ref[0, 0] = -mi / B


def kernel(input, target):
    x = input.reshape(B, N)
    y = target.reshape(B, N)
    parts = _sc_hist(x, y)
    parts4 = parts.reshape(NW, B, BINS, BINS)
    out = pl.pallas_call(
        _tc_mi_body,
        out_shape=jax.ShapeDtypeStruct((1, 1), jnp.float32),
        out_specs=pl.BlockSpec(memory_space=pltpu.SMEM),
    )(parts4)
    return out[0, 0]


# speedup vs baseline: 25.4418x; 1.6157x over previous
import functools

import jax
import jax.numpy as jnp
from jax import lax
from jax.experimental import pallas as pl
from jax.experimental.pallas import tpu as pltpu
from jax.experimental.pallas import tpu_sc as plsc

B = 8
N = 512 * 512
BINS = 32
NBINS2 = BINS * BINS

_info = plsc.get_sparse_core_info()
NC = _info.num_cores
NS = _info.num_subcores
NW = NC * NS
L = _info.num_lanes
CHUNK = N // NW


def _sc_hist_body(x_hbm, y_hbm, out_hbm, xv, yv, hist, semx, semy):
    wid = lax.axis_index("c") * NS + lax.axis_index("s")
    base = wid * CHUNK

    zeros = jnp.zeros((L,), jnp.float32)
    ones = jnp.ones((L,), jnp.float32)

    @plsc.parallel_loop(0, (B * NBINS2) // L, unroll=8)
    def _zero(i):
        hist[pl.ds(i * L, L)] = zeros

    cpx = pltpu.async_copy(x_hbm.at[0, pl.ds(base, CHUNK)], xv.at[0], semx)
    cpy = pltpu.async_copy(y_hbm.at[0, pl.ds(base, CHUNK)], yv.at[0], semy)
    for b in range(B):
        cur = b & 1
        cpx.wait()
        cpy.wait()
        if b + 1 < B:
            cpx = pltpu.async_copy(
                x_hbm.at[b + 1, pl.ds(base, CHUNK)], xv.at[1 - cur], semx
            )
            cpy = pltpu.async_copy(
                y_hbm.at[b + 1, pl.ds(base, CHUNK)], yv.at[1 - cur], semy
            )
        hbase = b * NBINS2

        @plsc.parallel_loop(0, CHUNK // L, unroll=8)
        def _points(i):
            xr = xv[cur, pl.ds(i * L, L)]
            yr = yv[cur, pl.ds(i * L, L)]
            vx = xr * 16.0 + 16.0
            vy = yr * 16.0 + 16.0
            valid = (vx >= 0.0) & (vx <= 32.0) & (vy >= 0.0) & (vy <= 32.0)
            ix = jnp.minimum(vx.astype(jnp.int32), BINS - 1)
            iy = jnp.minimum(vy.astype(jnp.int32), BINS - 1)
            flat = hbase + ix * BINS + iy
            plsc.addupdate_scatter(hist, [flat], ones, mask=valid)

    pltpu.sync_copy(hist, out_hbm.at[wid])


@functools.partial(
    pl.kernel,
    out_type=jax.ShapeDtypeStruct((NW, B * NBINS2), jnp.float32),
    mesh=plsc.VectorSubcoreMesh(core_axis_name="c", subcore_axis_name="s"),
    compiler_params=pltpu.CompilerParams(needs_layout_passes=False),
    scratch_types=[
        pltpu.VMEM((2, CHUNK), jnp.float32),
        pltpu.VMEM((2, CHUNK), jnp.float32),
        pltpu.VMEM((B * NBINS2,), jnp.float32),
        pltpu.SemaphoreType.DMA,
        pltpu.SemaphoreType.DMA,
    ],
)
def _sc_hist(x_hbm, y_hbm, out_hbm, xv, yv, hist, semx, semy):
    _sc_hist_body(x_hbm, y_hbm, out_hbm, xv, yv, hist, semx, semy)


def _tc_mi_body(parts_ref, out_ref):
    h = jnp.sum(parts_ref[...], axis=0)
    px = jnp.sum(h, axis=2)
    py = jnp.sum(h, axis=1)
    tot = jnp.sum(h, axis=(1, 2), keepdims=True)
    hn = h / tot
    pxn = px / jnp.sum(px, axis=1, keepdims=True)
    pyn = py / jnp.sum(py, axis=1, keepdims=True)
    pxy = pxn[:, :, None] * pyn[:, None, :]
    mask = (hn > 0) & (pxn[:, :, None] > 0) & (pyn[:, None, :] > 0)
    safe_ratio = jnp.where(mask, hn / jnp.where(mask, pxy, 1.0), 1.0)
    mi = jnp.sum(jnp.where(mask, hn * jnp.log(safe_ratio), 0.0))
    out_ref[0, 0] = -mi / B


def kernel(input, target):
    x = input.reshape(B, N)
    y = target.reshape(B, N)
    parts = _sc_hist(x, y)
    parts4 = parts.reshape(NW, B, BINS, BINS)
    out = pl.pallas_call(
        _tc_mi_body,
        out_shape=jax.ShapeDtypeStruct((1, 1), jnp.float32),
        out_specs=pl.BlockSpec(memory_space=pltpu.SMEM),
    )(parts4)
    return out[0, 0]


# speedup vs baseline: 28.0722x; 1.1034x over previous
import functools

import jax
import jax.numpy as jnp
from jax import lax
from jax.experimental import pallas as pl
from jax.experimental.pallas import tpu as pltpu
from jax.experimental.pallas import tpu_sc as plsc

B = 8
N = 512 * 512
BINS = 32
NBINS2 = BINS * BINS

_info = plsc.get_sparse_core_info()
NC = _info.num_cores
NS = _info.num_subcores
NW = NC * NS
L = _info.num_lanes
CHUNK = N // NW


def _sc_hist_body(x_hbm, y_hbm, out_hbm, xv, yv, hist, semx, semy):
    wid = lax.axis_index("c") * NS + lax.axis_index("s")
    base = wid * CHUNK

    zeros = jnp.zeros((L,), jnp.float32)
    ones = jnp.ones((L,), jnp.float32)

    for zb in range(B):

        @plsc.parallel_loop(0, NBINS2 // L, unroll=8)
        def _zero(i):
            hist[zb, pl.ds(i * L, L)] = zeros

    cpx = pltpu.async_copy(x_hbm.at[0, pl.ds(base, CHUNK)], xv.at[0], semx)
    cpy = pltpu.async_copy(y_hbm.at[0, pl.ds(base, CHUNK)], yv.at[0], semy)
    for b in range(B):
        cur = b & 1
        cpx.wait()
        cpy.wait()
        if b + 1 < B:
            cpx = pltpu.async_copy(
                x_hbm.at[b + 1, pl.ds(base, CHUNK)], xv.at[1 - cur], semx
            )
            cpy = pltpu.async_copy(
                y_hbm.at[b + 1, pl.ds(base, CHUNK)], yv.at[1 - cur], semy
            )
        bvec = jnp.full((L,), b, jnp.int32)

        @plsc.parallel_loop(0, CHUNK // L, unroll=8)
        def _points(i):
            xr = xv[cur, pl.ds(i * L, L)]
            yr = yv[cur, pl.ds(i * L, L)]
            vx = xr * 16.0 + 16.0
            vy = yr * 16.0 + 16.0
            valid = (jnp.minimum(vx, vy) >= 0.0) & (jnp.maximum(vx, vy) <= 32.0)
            tx = jnp.minimum(vx, 31.0).astype(jnp.int32)
            ty = jnp.minimum(vy, 31.0).astype(jnp.int32)
            flat = tx * BINS + ty
            plsc.addupdate_scatter(hist, [bvec, flat], ones, mask=valid)

    pltpu.sync_copy(hist, out_hbm.at[pl.ds(wid * B, B)])


@functools.partial(
    pl.kernel,
    out_type=jax.ShapeDtypeStruct((NW * B, NBINS2), jnp.float32),
    mesh=plsc.VectorSubcoreMesh(core_axis_name="c", subcore_axis_name="s"),
    compiler_params=pltpu.CompilerParams(needs_layout_passes=False),
    scratch_types=[
        pltpu.VMEM((2, CHUNK), jnp.float32),
        pltpu.VMEM((2, CHUNK), jnp.float32),
        pltpu.VMEM((B, NBINS2), jnp.float32),
        pltpu.SemaphoreType.DMA,
        pltpu.SemaphoreType.DMA,
    ],
)
def _sc_hist(x_hbm, y_hbm, out_hbm, xv, yv, hist, semx, semy):
    _sc_hist_body(x_hbm, y_hbm, out_hbm, xv, yv, hist, semx, semy)


def _tc_mi_body(parts_ref, out_ref):
    p3 = parts_ref[...].reshape(NW, B, NBINS2)
    h = jnp.sum(p3, axis=0)
    k = lax.broadcasted_iota(jnp.int32, (NBINS2, BINS), 0)
    c = lax.broadcasted_iota(jnp.int32, (NBINS2, BINS), 1)
    row_sel = (k // BINS == c).astype(jnp.float32)
    col_sel = (k % BINS == c).astype(jnp.float32)
    px = jax.lax.dot(h, row_sel)
    py = jax.lax.dot(h, col_sel)
    tot = jnp.sum(h, axis=1, keepdims=True)
    hn = h / tot
    pxn = px / jnp.sum(px, axis=1, keepdims=True)
    pyn = py / jnp.sum(py, axis=1, keepdims=True)
    pxe = jax.lax.dot(pxn, row_sel.T)
    pye = jax.lax.dot(pyn, col_sel.T)
    pxy = pxe * pye
    mask = (hn > 0) & (pxe > 0) & (pye > 0)
    safe_ratio = jnp.where(mask, hn / jnp.where(mask, pxy, 1.0), 1.0)
    mi = jnp.sum(jnp.where(mask, hn * jnp.log(safe_ratio), 0.0))
    out_ref[0, 0] = -mi / B


def kernel(input, target):
    x = input.reshape(B, N)
    y = target.reshape(B, N)
    parts = _sc_hist(x, y)
    out = pl.pallas_call(
        _tc_mi_body,
        out_shape=jax.ShapeDtypeStruct((1, 1), jnp.float32),
        out_specs=pl.BlockSpec(memory_space=pltpu.SMEM),
    )(parts)
    return out[0, 0]
